# baseline (speedup 1.0000x reference)
import functools

import jax
import jax.numpy as jnp
from jax import lax
from jax.experimental import pallas as pl
from jax.experimental.pallas import tpu as pltpu
from jax.experimental.pallas import tpu_sc as plsc

_B = 64
_N = 576
_NUM_MASK = 432
_TAIL = _N - _NUM_MASK

_ROT = ((13, 15, 26, 6), (17, 29, 16, 24))


def _threefry_keys():
    u32 = jnp.uint32
    row = jax.lax.broadcasted_iota(jnp.int32, (_B, _N), 0)
    col = jax.lax.broadcasted_iota(jnp.int32, (_B, _N), 1)
    x1 = (row * _N + col).astype(u32)
    x0 = jnp.zeros((_B, _N), u32)

    ks0 = u32(0)
    ks1 = u32(42)
    ks2 = u32(0 ^ 42 ^ 0x1BD11BDA)
    inj = ((ks1, ks2), (ks2, ks0), (ks0, ks1), (ks1, ks2), (ks2, ks0))

    x0 = x0 + ks0
    x1 = x1 + ks1
    for i in range(5):
        for r in _ROT[i % 2]:
            x0 = x0 + x1
            x1 = (x1 << u32(r)) | (x1 >> u32(32 - r))
            x1 = x0 ^ x1
        a, b = inj[i]
        x0 = x0 + a
        x1 = x1 + b + u32(i + 1)

    bits = x0 ^ x1
    return (bits >> u32(9)).astype(jnp.int32)


def _ranks_kernel(out_ref, keys_ref):
    keys_ref[:, :] = _threefry_keys()
    keys = keys_ref[:, :]
    ones_col = jnp.ones((_N, 1), jnp.float32)
    for j in range(_NUM_MASK, _N):
        t = keys_ref[:, j : j + 1]
        cmp = (keys < t).astype(jnp.float32)
        rank = jax.lax.dot_general(
            cmp, ones_col, (((1,), (0,)), ((), ())),
            preferred_element_type=jnp.float32,
        ).astype(jnp.int32)
        out_ref[:, j - _NUM_MASK : j - _NUM_MASK + 1] = rank


def _tail_ranks():
    return pl.pallas_call(
        _ranks_kernel,
        out_shape=jax.ShapeDtypeStruct((_B, _TAIL), jnp.int32),
        scratch_shapes=[pltpu.VMEM((_B, _N), jnp.int32)],
    )()


def _sc_scatter_mask(ranks):
    info = plsc.get_sparse_core_info()
    num_workers = info.num_cores * info.num_subcores
    rows_per_w = _B // num_workers
    num_cores = info.num_cores
    mesh = plsc.VectorSubcoreMesh(core_axis_name="c", subcore_axis_name="s")

    @functools.partial(
        pl.kernel,
        out_type=jax.ShapeDtypeStruct((_B, _N), jnp.int32),
        mesh=mesh,
        compiler_params=pltpu.CompilerParams(needs_layout_passes=False),
        scratch_types=[
            pltpu.VMEM((rows_per_w, _TAIL), jnp.int32),
            pltpu.VMEM((_N,), jnp.int32),
            pltpu.VMEM((_N,), jnp.int32),
        ],
    )
    def scatter_kernel(ranks_hbm, out_hbm, ranks_v, mask0_v, mask1_v):
        wid = lax.axis_index("s") * num_cores + lax.axis_index("c")
        base = wid * rows_per_w
        pltpu.sync_copy(ranks_hbm.at[pl.ds(base, rows_per_w)], ranks_v)
        ones16 = jnp.ones((16,), jnp.int32)
        zeros16 = jnp.zeros((16,), jnp.int32)
        for r, mask_v in enumerate((mask0_v, mask1_v)):
            for c in range(_N // 16):
                mask_v[pl.ds(16 * c, 16)] = ones16
            for k in range(_TAIL // 16):
                idx = ranks_v[r, pl.ds(16 * k, 16)]
                plsc.store_scatter(mask_v, [idx], zeros16)
            pltpu.sync_copy(mask_v, out_hbm.at[base + r])

    return scatter_kernel(ranks)


def kernel(x):
    assert x.shape[0] == _B
    ranks = _tail_ranks()
    return _sc_scatter_mask(ranks).astype(jnp.bool_)
